# baseline (speedup 1.0000x reference)
import math

import jax
import jax.numpy as jnp
from jax.experimental import pallas as pl
from jax.experimental.pallas import tpu as pltpu


def _router_body(x_ref, rw_ref, gate_ref, *, n_exp, scale):
    x = x_ref[...]
    rw = rw_ref[...]
    logits = jax.lax.dot_general(x, rw, (((1,), (1,)), ((), ())))
    i16 = jax.lax.broadcasted_iota(jnp.int32, logits.shape, 1)
    logits = jnp.where(i16 < n_exp, logits, -1e30)
    p = jax.nn.softmax(logits, axis=-1)
    v1 = jnp.max(p, axis=-1, keepdims=True)
    a1 = jnp.argmax(p, axis=-1)[:, None]
    m1 = i16 == a1
    p2 = jnp.where(m1, -1.0, p)
    v2 = jnp.max(p2, axis=-1, keepdims=True)
    a2 = jnp.argmax(p2, axis=-1)[:, None]
    m2 = i16 == a2
    g = jnp.where(m1, v1, 0.0) + jnp.where(m2, v2, 0.0)
    g = g + jnp.where(i16 == n_exp, 1.0, 0.0)
    gate_ref[...] = scale * g


def _moe_body(gate_ref, x_ref, w1_ref, b1_ref, w2_ref, b2_ref, out_ref, *, tt):
    t = pl.program_id(0)
    e = pl.program_id(1)
    x = x_ref[...]
    w1 = w1_ref[0]
    h = jax.lax.dot_general(x, w1, (((1,), (1,)), ((), ())),
                            preferred_element_type=jnp.float32)
    h = jax.nn.gelu(h + b1_ref[0])
    g16 = gate_ref[pl.ds(t * tt, tt), :]
    i16 = jax.lax.broadcasted_iota(jnp.int32, g16.shape, 1)
    gcol = jnp.sum(jnp.where(i16 == e, g16, 0.0), axis=1, keepdims=True)
    hb = (h * gcol).astype(jnp.bfloat16)
    w2 = w2_ref[0]
    y = jax.lax.dot_general(hb, w2, (((1,), (1,)), ((), ())),
                            preferred_element_type=jnp.float32)

    @pl.when(e == 0)
    def _init():
        out_ref[...] = jnp.zeros_like(out_ref)

    out_ref[...] += y + gcol * b2_ref[0]


def kernel(x, router_w, W1, b1, W2, b2, Ws1, bs1, Ws2, bs2):
    B, T, D = x.shape
    E, F, _ = W1.shape
    N = B * T
    NE = E + 1
    K = 2
    scale = 1.0 / math.sqrt(1.0 + K / E)

    xf = x.reshape(N, D)
    xbf = xf.astype(jnp.bfloat16)
    rw16 = jnp.zeros((16, D), jnp.float32).at[:E].set(router_w)
    W1c = jnp.concatenate([W1, Ws1[None]], axis=0).astype(jnp.bfloat16)
    b1c = jnp.concatenate([b1, bs1[None]], axis=0).reshape(NE, 1, F)
    W2c = jnp.concatenate([W2, Ws2[None]], axis=0).astype(jnp.bfloat16)
    b2c = jnp.concatenate([b2, bs2[None]], axis=0).reshape(NE, 1, D)

    RT = min(512, N)
    gate = pl.pallas_call(
        lambda xr, rr, gr: _router_body(xr, rr, gr, n_exp=E, scale=scale),
        grid=(N // RT,),
        in_specs=[
            pl.BlockSpec((RT, D), lambda t: (t, 0)),
            pl.BlockSpec((16, D), lambda t: (0, 0)),
        ],
        out_specs=pl.BlockSpec((RT, 16), lambda t: (t, 0)),
        out_shape=jax.ShapeDtypeStruct((N, 16), jnp.float32),
    )(xf, rw16)

    TT = min(512, N)
    out = pl.pallas_call(
        lambda gr, xr, w1r, b1r, w2r, b2r, orf: _moe_body(
            gr, xr, w1r, b1r, w2r, b2r, orf, tt=TT),
        grid=(N // TT, NE),
        in_specs=[
            pl.BlockSpec((N, 16), lambda t, e: (0, 0)),
            pl.BlockSpec((TT, D), lambda t, e: (t, 0)),
            pl.BlockSpec((1, F, D), lambda t, e: (e, 0, 0)),
            pl.BlockSpec((1, 1, F), lambda t, e: (e, 0, 0)),
            pl.BlockSpec((1, D, F), lambda t, e: (e, 0, 0)),
            pl.BlockSpec((1, 1, D), lambda t, e: (e, 0, 0)),
        ],
        out_specs=pl.BlockSpec((TT, D), lambda t, e: (t, 0)),
        out_shape=jax.ShapeDtypeStruct((N, D), jnp.float32),
        compiler_params=pltpu.CompilerParams(
            dimension_semantics=("parallel", "arbitrary")),
    )(gate, xbf, W1c, b1c, W2c, b2c)

    return out.reshape(B, T, D)


# speedup vs baseline: 1.6625x; 1.6625x over previous
import functools
import math

import jax
import jax.numpy as jnp
from jax import lax
from jax.experimental import pallas as pl
from jax.experimental.pallas import tpu as pltpu
from jax.experimental.pallas import tpu_sc as plsc

_NC, _NS, _NW = 2, 16, 32


def _excl_cumsum_rows(a):
    n = a.shape[0]
    s = jnp.concatenate([jnp.zeros_like(a[:1]), a[:-1]], axis=0)
    d = 1
    while d < n:
        s = s + jnp.concatenate([jnp.zeros_like(s[:d]), s[:-d]], axis=0)
        d *= 2
    return s


def _excl_cumsum_lanes(a):
    n = a.shape[1]
    s = jnp.concatenate([jnp.zeros_like(a[:, :1]), a[:, :-1]], axis=1)
    d = 1
    while d < n:
        s = s + jnp.concatenate([jnp.zeros_like(s[:, :d]), s[:, :-d]], axis=1)
        d *= 2
    return s


def _sel_lane(m, a, i16):
    return jnp.sum(jnp.where(i16 == a, m, 0.0), axis=1, keepdims=True)


def _route_body(x_ref, rw_ref, post_ref, valt_ref, misc_ref, *,
                n_exp, scale, bt, nbmax):
    x = x_ref[...]
    rw = rw_ref[...]
    logits = lax.dot_general(x, rw, (((1,), (1,)), ((), ())))
    i16 = lax.broadcasted_iota(jnp.int32, logits.shape, 1)
    logits = jnp.where(i16 < n_exp, logits, -1e30)
    p = jax.nn.softmax(logits, axis=-1)
    v1 = jnp.max(p, axis=-1, keepdims=True)
    a1 = jnp.argmax(p, axis=-1)[:, None]
    m1 = i16 == a1
    p2 = jnp.where(m1, -1.0, p)
    v2 = jnp.max(p2, axis=-1, keepdims=True)
    a2 = jnp.argmax(p2, axis=-1)[:, None]
    m2 = i16 == a2

    oh1 = m1.astype(jnp.float32)
    oh2 = m2.astype(jnp.float32)
    c1 = _excl_cumsum_rows(oh1)
    c2 = _excl_cumsum_rows(oh2)
    cnt1 = jnp.sum(oh1, axis=0, keepdims=True)
    cnt2 = jnp.sum(oh2, axis=0, keepdims=True)
    g = cnt1 + cnt2
    nb = jnp.ceil(g / float(bt))
    nboff = _excl_cumsum_lanes(nb)
    off = nboff * float(bt)

    pos0 = _sel_lane(off + c1, a1, i16)
    pos1 = _sel_lane(off + cnt1 + c2, a2, i16)
    post = jnp.concatenate([pos0, pos1], axis=1).astype(jnp.int32)
    post_ref[...] = jnp.transpose(post, (1, 0))
    valt = scale * jnp.concatenate([v1, v2], axis=1)
    valt_ref[...] = jnp.transpose(valt, (1, 0))

    r16 = lax.broadcasted_iota(jnp.int32, (16, 16), 0)
    l16 = lax.broadcasted_iota(jnp.int32, (16, 16), 1)
    nboff_col = jnp.sum(jnp.where(r16 == l16,
                                  jnp.broadcast_to(nboff, (16, 16)), 0.0),
                        axis=1, keepdims=True)
    bi = lax.broadcasted_iota(jnp.int32, (16, 128), 1)
    r128 = lax.broadcasted_iota(jnp.int32, (16, 128), 0)
    cmp = (bi >= nboff_col.astype(jnp.int32)) & (r128 < n_exp)
    be = jnp.sum(cmp.astype(jnp.int32), axis=0, keepdims=True) - 1
    be = jnp.clip(be, 0, n_exp - 1)
    nbt = jnp.sum(nb).astype(jnp.int32)
    l128 = lax.broadcasted_iota(jnp.int32, (1, 128), 1)
    misc = jnp.where(l128 < nbmax, be, 0)
    misc_ref[...] = jnp.where(l128 == 64, nbt, misc)


def _shared_body(x_ref, w1_ref, b1_ref, w2_ref, b2_ref, o_ref, *, scale):
    f = pl.program_id(1)
    x = x_ref[...]
    h = lax.dot_general(x, w1_ref[...], (((1,), (1,)), ((), ())),
                        preferred_element_type=jnp.float32)
    h = jax.nn.gelu(h + b1_ref[...]) * scale
    y = lax.dot_general(h, w2_ref[...], (((1,), (1,)), ((), ())),
                        preferred_element_type=jnp.float32)

    @pl.when(f == 0)
    def _init():
        o_ref[...] = jnp.zeros_like(o_ref) + scale * b2_ref[...]

    o_ref[...] += y


def _expert_body(be_ref, nbt_ref, xs_ref, w1_ref, b1_ref, w2_ref, b2_ref,
                 vs_ref, y_ref):
    b = pl.program_id(0)
    f = pl.program_id(1)

    @pl.when(b < nbt_ref[0])
    def _go():
        x = xs_ref[...]
        h = lax.dot_general(x, w1_ref[0], (((1,), (1,)), ((), ())),
                            preferred_element_type=jnp.float32)
        h = jax.nn.gelu(h + b1_ref[0])
        vcol = jnp.transpose(vs_ref[0], (1, 0))
        hs = h * vcol
        y = lax.dot_general(hs, w2_ref[0], (((1,), (1,)), ((), ())),
                            preferred_element_type=jnp.float32)

        @pl.when(f == 0)
        def _init():
            y_ref[...] = jnp.zeros_like(y_ref) + vcol * b2_ref[0]

        y_ref[...] += y


def kernel(x, router_w, W1, b1, W2, b2, Ws1, bs1, Ws2, bs2):
    B, T, D = x.shape
    E, F, _ = W1.shape
    N = B * T
    K = 2
    scale = 1.0 / math.sqrt(1.0 + K / E)
    BT = 512 if N >= 4096 else 128
    NBMAX = (N * K) // BT + E
    PEXP = NBMAX * BT
    FB = F // 2

    xf = x.reshape(N, D)
    rw16 = jnp.zeros((16, D), jnp.float32).at[:E].set(router_w)

    post, valt, misc = pl.pallas_call(
        functools.partial(_route_body, n_exp=E, scale=scale, bt=BT,
                          nbmax=NBMAX),
        grid=(1,),
        in_specs=[
            pl.BlockSpec((N, D), lambda i: (0, 0)),
            pl.BlockSpec((16, D), lambda i: (0, 0)),
        ],
        out_specs=[
            pl.BlockSpec((2, N), lambda i: (0, 0)),
            pl.BlockSpec((2, N), lambda i: (0, 0)),
            pl.BlockSpec((1, 128), lambda i: (0, 0)),
        ],
        out_shape=[
            jax.ShapeDtypeStruct((2, N), jnp.int32),
            jax.ShapeDtypeStruct((2, N), jnp.float32),
            jax.ShapeDtypeStruct((1, 128), jnp.int32),
        ],
    )(xf, rw16)

    mesh = plsc.VectorSubcoreMesh(core_axis_name="c", subcore_axis_name="s")
    tpn = N // _NW

    @functools.partial(
        pl.kernel,
        out_type=(jax.ShapeDtypeStruct((PEXP,), jnp.int32),
                  jax.ShapeDtypeStruct((PEXP,), jnp.float32)),
        mesh=mesh,
        scratch_types=[pltpu.VMEM((tpn,), jnp.int32),
                       pltpu.VMEM((tpn,), jnp.int32),
                       pltpu.VMEM((tpn,), jnp.float32)],
    )
    def _sc_scatter(post_h, valt_h, trow_h, vsort_h, idx_v, tid_v, val_v):
        wid = lax.axis_index("s") * _NC + lax.axis_index("c")
        base = wid * tpn
        for c in range(tpn // 16):
            tid_v[pl.ds(c * 16, 16)] = (lax.iota(jnp.int32, 16)
                                        + (base + c * 16))
        for k in range(2):
            pltpu.sync_copy(post_h.at[k, pl.ds(base, tpn)], idx_v)
            pltpu.sync_copy(valt_h.at[k, pl.ds(base, tpn)], val_v)
            pltpu.sync_copy(tid_v, trow_h.at[idx_v])
            pltpu.sync_copy(val_v, vsort_h.at[idx_v])

    trow, vsort = _sc_scatter(post, valt)

    spw = PEXP // _NW
    CH = min(64, tpn)

    @functools.partial(
        pl.kernel,
        out_type=jax.ShapeDtypeStruct((PEXP, D), jnp.float32),
        mesh=mesh,
        scratch_types=[pltpu.VMEM((CH,), jnp.int32),
                       pltpu.VMEM((CH, D), jnp.float32),
                       pltpu.SemaphoreType.DMA],
    )
    def _sc_gather(trow_h, x_h, xs_h, idx_v, rows_v, sem):
        wid = lax.axis_index("s") * _NC + lax.axis_index("c")
        for ci in range(spw // CH):
            base = wid * spw + ci * CH
            pltpu.sync_copy(trow_h.at[pl.ds(base, CH)], idx_v)
            for c in range(CH // 16):
                v = idx_v[pl.ds(c * 16, 16)]
                idx_v[pl.ds(c * 16, 16)] = jnp.clip(v, 0, N - 1)
            pltpu.async_copy(x_h.at[idx_v], rows_v, sem).wait()
            pltpu.sync_copy(rows_v, xs_h.at[pl.ds(base, CH)])

    xs = _sc_gather(trow, xf)

    ysh = pl.pallas_call(
        functools.partial(_shared_body, scale=scale),
        grid=(N // BT, 2),
        in_specs=[
            pl.BlockSpec((BT, D), lambda t, f: (t, 0)),
            pl.BlockSpec((FB, D), lambda t, f: (f, 0)),
            pl.BlockSpec((1, FB), lambda t, f: (0, f)),
            pl.BlockSpec((D, FB), lambda t, f: (0, f)),
            pl.BlockSpec((1, D), lambda t, f: (0, 0)),
        ],
        out_specs=pl.BlockSpec((BT, D), lambda t, f: (t, 0)),
        out_shape=jax.ShapeDtypeStruct((N, D), jnp.float32),
        compiler_params=pltpu.CompilerParams(
            dimension_semantics=("arbitrary", "arbitrary")),
    )(xf, Ws1, bs1.reshape(1, F), Ws2, bs2.reshape(1, D))

    be_arr = misc[0, :NBMAX]
    nbt_arr = misc[0, 64:65]
    yexp = pl.pallas_call(
        _expert_body,
        grid_spec=pltpu.PrefetchScalarGridSpec(
            num_scalar_prefetch=2,
            grid=(NBMAX, 2),
            in_specs=[
                pl.BlockSpec((BT, D), lambda b, f, be, nbt: (b, 0)),
                pl.BlockSpec((1, FB, D), lambda b, f, be, nbt: (be[b], f, 0)),
                pl.BlockSpec((1, 1, FB), lambda b, f, be, nbt: (be[b], 0, f)),
                pl.BlockSpec((1, D, FB), lambda b, f, be, nbt: (be[b], 0, f)),
                pl.BlockSpec((1, 1, D), lambda b, f, be, nbt: (be[b], 0, 0)),
                pl.BlockSpec((1, 1, BT), lambda b, f, be, nbt: (b, 0, 0)),
            ],
            out_specs=pl.BlockSpec((BT, D), lambda b, f, be, nbt: (b, 0)),
        ),
        out_shape=jax.ShapeDtypeStruct((PEXP, D), jnp.float32),
        compiler_params=pltpu.CompilerParams(
            dimension_semantics=("arbitrary", "arbitrary")),
    )(be_arr, nbt_arr, xs, W1, b1.reshape(E, 1, F), W2,
      b2.reshape(E, 1, D), vsort.reshape(NBMAX, 1, BT))

    C3 = min(32, tpn)

    @functools.partial(
        pl.kernel,
        out_type=jax.ShapeDtypeStruct((N, D), jnp.float32),
        mesh=mesh,
        scratch_types=[pltpu.VMEM((C3, D), jnp.float32),
                       pltpu.VMEM((C3, D), jnp.float32),
                       pltpu.VMEM((C3, D), jnp.float32),
                       pltpu.VMEM((C3,), jnp.int32),
                       pltpu.VMEM((C3,), jnp.int32),
                       pltpu.SemaphoreType.DMA],
    )
    def _sc_combine(ysh_h, yexp_h, post_h, out_h, bsh, b0, b1, i0, i1, sem):
        wid = lax.axis_index("s") * _NC + lax.axis_index("c")
        for ci in range(tpn // C3):
            base = wid * tpn + ci * C3
            pltpu.sync_copy(ysh_h.at[pl.ds(base, C3)], bsh)
            pltpu.sync_copy(post_h.at[0, pl.ds(base, C3)], i0)
            pltpu.sync_copy(post_h.at[1, pl.ds(base, C3)], i1)
            pltpu.async_copy(yexp_h.at[i0], b0, sem).wait()
            pltpu.async_copy(yexp_h.at[i1], b1, sem).wait()

            def _vadd(j, carry):
                r = j // (D // 16)
                s = pl.ds((j % (D // 16)) * 16, 16)
                bsh[r, s] = bsh[r, s] + b0[r, s] + b1[r, s]
                return carry

            lax.fori_loop(0, C3 * (D // 16), _vadd, 0)
            pltpu.sync_copy(bsh, out_h.at[pl.ds(base, C3)])

    out = _sc_combine(ysh, yexp, post)
    return out.reshape(B, T, D)


# speedup vs baseline: 2.3190x; 1.3949x over previous
import functools
import math

import jax
import jax.numpy as jnp
from jax import lax
from jax.experimental import pallas as pl
from jax.experimental.pallas import tpu as pltpu
from jax.experimental.pallas import tpu_sc as plsc

_NC, _NS, _NW = 2, 16, 32


def _excl_cumsum_rows(a):
    n = a.shape[0]
    s = jnp.concatenate([jnp.zeros_like(a[:1]), a[:-1]], axis=0)
    d = 1
    while d < n:
        s = s + jnp.concatenate([jnp.zeros_like(s[:d]), s[:-d]], axis=0)
        d *= 2
    return s


def _excl_cumsum_lanes(a):
    n = a.shape[1]
    s = jnp.concatenate([jnp.zeros_like(a[:, :1]), a[:, :-1]], axis=1)
    d = 1
    while d < n:
        s = s + jnp.concatenate([jnp.zeros_like(s[:, :d]), s[:, :-d]], axis=1)
        d *= 2
    return s


def _sel_lane(m, a, i16):
    return jnp.sum(jnp.where(i16 == a, m, 0.0), axis=1, keepdims=True)


def _route_body(x_ref, rw_ref, post_ref, valt_ref, misc_ref, *,
                n_exp, scale, bt, nbmax):
    x = x_ref[...]
    rw = rw_ref[...]
    logits = lax.dot_general(x, rw, (((1,), (1,)), ((), ())))
    i16 = lax.broadcasted_iota(jnp.int32, logits.shape, 1)
    logits = jnp.where(i16 < n_exp, logits, -1e30)
    p = jax.nn.softmax(logits, axis=-1)
    v1 = jnp.max(p, axis=-1, keepdims=True)
    a1 = jnp.argmax(p, axis=-1)[:, None]
    m1 = i16 == a1
    p2 = jnp.where(m1, -1.0, p)
    v2 = jnp.max(p2, axis=-1, keepdims=True)
    a2 = jnp.argmax(p2, axis=-1)[:, None]
    m2 = i16 == a2

    oh1 = m1.astype(jnp.float32)
    oh2 = m2.astype(jnp.float32)
    c1 = _excl_cumsum_rows(oh1)
    c2 = _excl_cumsum_rows(oh2)
    cnt1 = jnp.sum(oh1, axis=0, keepdims=True)
    cnt2 = jnp.sum(oh2, axis=0, keepdims=True)
    g = cnt1 + cnt2
    nb = jnp.ceil(g / float(bt))
    nboff = _excl_cumsum_lanes(nb)
    off = nboff * float(bt)

    pos0 = _sel_lane(off + c1, a1, i16)
    pos1 = _sel_lane(off + cnt1 + c2, a2, i16)
    post = jnp.concatenate([pos0, pos1], axis=1).astype(jnp.int32)
    post_ref[...] = jnp.transpose(post, (1, 0))
    valt = scale * jnp.concatenate([v1, v2], axis=1)
    valt_ref[...] = jnp.transpose(valt, (1, 0))

    r16 = lax.broadcasted_iota(jnp.int32, (16, 16), 0)
    l16 = lax.broadcasted_iota(jnp.int32, (16, 16), 1)
    nboff_col = jnp.sum(jnp.where(r16 == l16,
                                  jnp.broadcast_to(nboff, (16, 16)), 0.0),
                        axis=1, keepdims=True)
    bi = lax.broadcasted_iota(jnp.int32, (16, 128), 1)
    r128 = lax.broadcasted_iota(jnp.int32, (16, 128), 0)
    cmp = (bi >= nboff_col.astype(jnp.int32)) & (r128 < n_exp)
    be = jnp.sum(cmp.astype(jnp.int32), axis=0, keepdims=True) - 1
    be = jnp.clip(be, 0, n_exp - 1)
    nbt = jnp.sum(nb).astype(jnp.int32)
    l128 = lax.broadcasted_iota(jnp.int32, (1, 128), 1)
    misc = jnp.where(l128 < nbmax, be, 0)
    misc_ref[...] = jnp.where(l128 == 64, nbt, misc)


def _shared_body(x_ref, w1_ref, b1_ref, w2_ref, b2_ref, o_ref, *, scale):
    f = pl.program_id(1)
    x = x_ref[...]
    h = lax.dot_general(x, w1_ref[...], (((1,), (1,)), ((), ())),
                        preferred_element_type=jnp.float32)
    h = jax.nn.gelu(h + b1_ref[...]) * scale
    y = lax.dot_general(h, w2_ref[...], (((1,), (1,)), ((), ())),
                        preferred_element_type=jnp.float32)

    @pl.when(f == 0)
    def _init():
        o_ref[...] = jnp.zeros_like(o_ref) + scale * b2_ref[...]

    o_ref[...] += y


def _expert_body(be_ref, nbt_ref, xs_ref, w1_ref, b1_ref, w2_ref, b2_ref,
                 vs_ref, y_ref):
    b = pl.program_id(0)
    f = pl.program_id(1)

    @pl.when(b < nbt_ref[0])
    def _go():
        x = xs_ref[...]
        h = lax.dot_general(x, w1_ref[0], (((1,), (1,)), ((), ())),
                            preferred_element_type=jnp.float32)
        h = jax.nn.gelu(h + b1_ref[0])
        vcol = jnp.transpose(vs_ref[0], (1, 0))
        hs = h * vcol
        y = lax.dot_general(hs, w2_ref[0], (((1,), (1,)), ((), ())),
                            preferred_element_type=jnp.float32)

        @pl.when(f == 0)
        def _init():
            y_ref[...] = jnp.zeros_like(y_ref) + vcol * b2_ref[0]

        y_ref[...] += y


def kernel(x, router_w, W1, b1, W2, b2, Ws1, bs1, Ws2, bs2):
    B, T, D = x.shape
    E, F, _ = W1.shape
    N = B * T
    K = 2
    scale = 1.0 / math.sqrt(1.0 + K / E)
    BT = 512 if N >= 4096 else 128
    NBMAX = (N * K) // BT + E
    PEXP = NBMAX * BT
    FB = F // 2

    xf = x.reshape(N, D)
    rw16 = jnp.zeros((16, D), jnp.float32).at[:E].set(router_w)

    post, valt, misc = pl.pallas_call(
        functools.partial(_route_body, n_exp=E, scale=scale, bt=BT,
                          nbmax=NBMAX),
        grid=(1,),
        in_specs=[
            pl.BlockSpec((N, D), lambda i: (0, 0)),
            pl.BlockSpec((16, D), lambda i: (0, 0)),
        ],
        out_specs=[
            pl.BlockSpec((2, N), lambda i: (0, 0)),
            pl.BlockSpec((2, N), lambda i: (0, 0)),
            pl.BlockSpec((1, 128), lambda i: (0, 0)),
        ],
        out_shape=[
            jax.ShapeDtypeStruct((2, N), jnp.int32),
            jax.ShapeDtypeStruct((2, N), jnp.float32),
            jax.ShapeDtypeStruct((1, 128), jnp.int32),
        ],
    )(xf, rw16)

    mesh = plsc.VectorSubcoreMesh(core_axis_name="c", subcore_axis_name="s")
    tpn = N // _NW

    CH = min(64, tpn)
    NCH = tpn // CH
    _disp_scratch = ([pltpu.VMEM((CH, D), jnp.float32)]
                     + [pltpu.VMEM((CH,), jnp.int32) for _ in range(2 * NCH)]
                     + [pltpu.VMEM((CH,), jnp.float32) for _ in range(2 * NCH)])

    @functools.partial(
        pl.kernel,
        out_type=(jax.ShapeDtypeStruct((PEXP, D), jnp.float32),
                  jax.ShapeDtypeStruct((PEXP,), jnp.float32)),
        mesh=mesh,
        scratch_types=_disp_scratch,
    )
    def _sc_dispatch(post_h, valt_h, x_h, xs_h, vs_h, rows_v, *bufs):
        pb = bufs[:2 * NCH]
        vb = bufs[2 * NCH:]
        wid = lax.axis_index("s") * _NC + lax.axis_index("c")
        base = wid * tpn
        for k in range(2):
            for ci in range(NCH):
                j = k * NCH + ci
                sl = pl.ds(base + ci * CH, CH)
                pltpu.sync_copy(post_h.at[k, sl], pb[j])
                pltpu.sync_copy(valt_h.at[k, sl], vb[j])
                pltpu.sync_copy(vb[j], vs_h.at[pb[j]])
        for ci in range(NCH):
            pltpu.sync_copy(x_h.at[pl.ds(base + ci * CH, CH)], rows_v)
            pltpu.sync_copy(rows_v, xs_h.at[pb[ci]])
            pltpu.sync_copy(rows_v, xs_h.at[pb[NCH + ci]])

    xs, vsort = _sc_dispatch(post, valt, xf)

    ysh = pl.pallas_call(
        functools.partial(_shared_body, scale=scale),
        grid=(N // BT, 2),
        in_specs=[
            pl.BlockSpec((BT, D), lambda t, f: (t, 0)),
            pl.BlockSpec((FB, D), lambda t, f: (f, 0)),
            pl.BlockSpec((1, FB), lambda t, f: (0, f)),
            pl.BlockSpec((D, FB), lambda t, f: (0, f)),
            pl.BlockSpec((1, D), lambda t, f: (0, 0)),
        ],
        out_specs=pl.BlockSpec((BT, D), lambda t, f: (t, 0)),
        out_shape=jax.ShapeDtypeStruct((N, D), jnp.float32),
        compiler_params=pltpu.CompilerParams(
            dimension_semantics=("arbitrary", "arbitrary")),
    )(xf, Ws1, bs1.reshape(1, F), Ws2, bs2.reshape(1, D))

    be_arr = misc[0, :NBMAX]
    nbt_arr = misc[0, 64:65]
    yexp = pl.pallas_call(
        _expert_body,
        grid_spec=pltpu.PrefetchScalarGridSpec(
            num_scalar_prefetch=2,
            grid=(NBMAX, 2),
            in_specs=[
                pl.BlockSpec((BT, D), lambda b, f, be, nbt: (b, 0)),
                pl.BlockSpec((1, FB, D), lambda b, f, be, nbt: (be[b], f, 0)),
                pl.BlockSpec((1, 1, FB), lambda b, f, be, nbt: (be[b], 0, f)),
                pl.BlockSpec((1, D, FB), lambda b, f, be, nbt: (be[b], 0, f)),
                pl.BlockSpec((1, 1, D), lambda b, f, be, nbt: (be[b], 0, 0)),
                pl.BlockSpec((1, 1, BT), lambda b, f, be, nbt: (b, 0, 0)),
            ],
            out_specs=pl.BlockSpec((BT, D), lambda b, f, be, nbt: (b, 0)),
        ),
        out_shape=jax.ShapeDtypeStruct((PEXP, D), jnp.float32),
        compiler_params=pltpu.CompilerParams(
            dimension_semantics=("arbitrary", "arbitrary")),
    )(be_arr, nbt_arr, xs, W1, b1.reshape(E, 1, F), W2,
      b2.reshape(E, 1, D), vsort.reshape(NBMAX, 1, BT))

    C3 = min(32, tpn)

    @functools.partial(
        pl.kernel,
        out_type=jax.ShapeDtypeStruct((N, D), jnp.float32),
        mesh=mesh,
        scratch_types=[pltpu.VMEM((C3, D), jnp.float32),
                       pltpu.VMEM((C3, D), jnp.float32),
                       pltpu.VMEM((C3, D), jnp.float32),
                       pltpu.VMEM((C3,), jnp.int32),
                       pltpu.VMEM((C3,), jnp.int32),
                       pltpu.SemaphoreType.DMA],
    )
    def _sc_combine(ysh_h, yexp_h, post_h, out_h, bsh, b0, b1, i0, i1, sem):
        wid = lax.axis_index("s") * _NC + lax.axis_index("c")
        for ci in range(tpn // C3):
            base = wid * tpn + ci * C3
            pltpu.sync_copy(ysh_h.at[pl.ds(base, C3)], bsh)
            pltpu.sync_copy(post_h.at[0, pl.ds(base, C3)], i0)
            pltpu.sync_copy(post_h.at[1, pl.ds(base, C3)], i1)
            pltpu.async_copy(yexp_h.at[i0], b0, sem).wait()
            pltpu.async_copy(yexp_h.at[i1], b1, sem).wait()

            def _vadd_row(r, carry):
                for c in range(D // 16):
                    s = pl.ds(c * 16, 16)
                    bsh[r, s] = bsh[r, s] + b0[r, s] + b1[r, s]
                return carry

            lax.fori_loop(0, C3, _vadd_row, 0)
            pltpu.sync_copy(bsh, out_h.at[pl.ds(base, C3)])

    out = _sc_combine(ysh, yexp, post)
    return out.reshape(B, T, D)


# speedup vs baseline: 2.3285x; 1.0041x over previous
import functools
import math

import jax
import jax.numpy as jnp
from jax import lax
from jax.experimental import pallas as pl
from jax.experimental.pallas import tpu as pltpu
from jax.experimental.pallas import tpu_sc as plsc

_NC, _NS, _NW = 2, 16, 32


def _excl_cumsum_rows(a):
    n = a.shape[0]
    s = jnp.concatenate([jnp.zeros_like(a[:1]), a[:-1]], axis=0)
    d = 1
    while d < n:
        s = s + jnp.concatenate([jnp.zeros_like(s[:d]), s[:-d]], axis=0)
        d *= 2
    return s


def _excl_cumsum_lanes(a):
    n = a.shape[1]
    s = jnp.concatenate([jnp.zeros_like(a[:, :1]), a[:, :-1]], axis=1)
    d = 1
    while d < n:
        s = s + jnp.concatenate([jnp.zeros_like(s[:, :d]), s[:, :-d]], axis=1)
        d *= 2
    return s


def _sel_lane(m, a, i16):
    return jnp.sum(jnp.where(i16 == a, m, 0.0), axis=1, keepdims=True)


def _route_body(x_ref, rw_ref, post_ref, valt_ref, misc_ref, *,
                n_exp, scale, bt, nbmax):
    x = x_ref[...]
    rw = rw_ref[...]
    logits = lax.dot_general(x, rw, (((1,), (1,)), ((), ())))
    i16 = lax.broadcasted_iota(jnp.int32, logits.shape, 1)
    logits = jnp.where(i16 < n_exp, logits, -1e30)
    p = jax.nn.softmax(logits, axis=-1)
    v1 = jnp.max(p, axis=-1, keepdims=True)
    a1 = jnp.argmax(p, axis=-1)[:, None]
    m1 = i16 == a1
    p2 = jnp.where(m1, -1.0, p)
    v2 = jnp.max(p2, axis=-1, keepdims=True)
    a2 = jnp.argmax(p2, axis=-1)[:, None]
    m2 = i16 == a2

    oh1 = m1.astype(jnp.float32)
    oh2 = m2.astype(jnp.float32)
    c1 = _excl_cumsum_rows(oh1)
    c2 = _excl_cumsum_rows(oh2)
    cnt1 = jnp.sum(oh1, axis=0, keepdims=True)
    cnt2 = jnp.sum(oh2, axis=0, keepdims=True)
    g = cnt1 + cnt2
    nb = jnp.ceil(g / float(bt))
    nboff = _excl_cumsum_lanes(nb)
    off = nboff * float(bt)

    pos0 = _sel_lane(off + c1, a1, i16)
    pos1 = _sel_lane(off + cnt1 + c2, a2, i16)
    post = jnp.concatenate([pos0, pos1], axis=1).astype(jnp.int32)
    post_ref[...] = jnp.transpose(post, (1, 0))
    valt = scale * jnp.concatenate([v1, v2], axis=1)
    valt_ref[...] = jnp.transpose(valt, (1, 0))

    r16 = lax.broadcasted_iota(jnp.int32, (16, 16), 0)
    l16 = lax.broadcasted_iota(jnp.int32, (16, 16), 1)
    nboff_col = jnp.sum(jnp.where(r16 == l16,
                                  jnp.broadcast_to(nboff, (16, 16)), 0.0),
                        axis=1, keepdims=True)
    bi = lax.broadcasted_iota(jnp.int32, (16, 128), 1)
    r128 = lax.broadcasted_iota(jnp.int32, (16, 128), 0)
    cmp = (bi >= nboff_col.astype(jnp.int32)) & (r128 < n_exp)
    be = jnp.sum(cmp.astype(jnp.int32), axis=0, keepdims=True) - 1
    be = jnp.clip(be, 0, n_exp - 1)
    nbt = jnp.sum(nb).astype(jnp.int32)
    l128 = lax.broadcasted_iota(jnp.int32, (1, 128), 1)
    misc = jnp.where(l128 < nbmax, be, 0)
    misc_ref[...] = jnp.where(l128 == 64, nbt, misc)


def _shared_body(x_ref, w1_ref, b1_ref, w2_ref, b2_ref, o_ref, *, scale):
    f = pl.program_id(1)
    x = x_ref[...]
    h = lax.dot_general(x, w1_ref[...], (((1,), (1,)), ((), ())),
                        preferred_element_type=jnp.float32)
    h = jax.nn.gelu(h + b1_ref[...]) * scale
    y = lax.dot_general(h, w2_ref[...], (((1,), (1,)), ((), ())),
                        preferred_element_type=jnp.float32)

    @pl.when(f == 0)
    def _init():
        o_ref[...] = jnp.zeros_like(o_ref) + scale * b2_ref[...]

    o_ref[...] += y


def _expert_body(be_ref, nbt_ref, xs_ref, w1_ref, b1_ref, w2_ref, b2_ref,
                 vs_ref, y_ref):
    b = pl.program_id(0)
    f = pl.program_id(1)

    @pl.when(b < nbt_ref[0])
    def _go():
        x = xs_ref[...]
        h = lax.dot_general(x, w1_ref[0], (((1,), (1,)), ((), ())),
                            preferred_element_type=jnp.float32)
        h = jax.nn.gelu(h + b1_ref[0])
        vcol = jnp.transpose(vs_ref[0], (1, 0))
        hs = h * vcol
        y = lax.dot_general(hs, w2_ref[0], (((1,), (1,)), ((), ())),
                            preferred_element_type=jnp.float32)

        @pl.when(f == 0)
        def _init():
            y_ref[...] = jnp.zeros_like(y_ref) + vcol * b2_ref[0]

        y_ref[...] += y


def kernel(x, router_w, W1, b1, W2, b2, Ws1, bs1, Ws2, bs2):
    B, T, D = x.shape
    E, F, _ = W1.shape
    N = B * T
    K = 2
    scale = 1.0 / math.sqrt(1.0 + K / E)
    BT = 512 if N >= 4096 else 128
    NBMAX = (N * K) // BT + E
    PEXP = NBMAX * BT
    FB = F // 2

    xf = x.reshape(N, D)
    rw16 = jnp.zeros((16, D), jnp.float32).at[:E].set(router_w)

    post, valt, misc = pl.pallas_call(
        functools.partial(_route_body, n_exp=E, scale=scale, bt=BT,
                          nbmax=NBMAX),
        grid=(1,),
        in_specs=[
            pl.BlockSpec((N, D), lambda i: (0, 0)),
            pl.BlockSpec((16, D), lambda i: (0, 0)),
        ],
        out_specs=[
            pl.BlockSpec((2, N), lambda i: (0, 0)),
            pl.BlockSpec((2, N), lambda i: (0, 0)),
            pl.BlockSpec((1, 128), lambda i: (0, 0)),
        ],
        out_shape=[
            jax.ShapeDtypeStruct((2, N), jnp.int32),
            jax.ShapeDtypeStruct((2, N), jnp.float32),
            jax.ShapeDtypeStruct((1, 128), jnp.int32),
        ],
    )(xf, rw16)

    ysh = pl.pallas_call(
        functools.partial(_shared_body, scale=scale),
        grid=(N // BT, 2),
        in_specs=[
            pl.BlockSpec((BT, D), lambda t, f: (t, 0)),
            pl.BlockSpec((FB, D), lambda t, f: (f, 0)),
            pl.BlockSpec((1, FB), lambda t, f: (0, f)),
            pl.BlockSpec((D, FB), lambda t, f: (0, f)),
            pl.BlockSpec((1, D), lambda t, f: (0, 0)),
        ],
        out_specs=pl.BlockSpec((BT, D), lambda t, f: (t, 0)),
        out_shape=jax.ShapeDtypeStruct((N, D), jnp.float32),
        compiler_params=pltpu.CompilerParams(
            dimension_semantics=("arbitrary", "arbitrary")),
    )(xf, Ws1, bs1.reshape(1, F), Ws2, bs2.reshape(1, D))

    mesh = plsc.VectorSubcoreMesh(core_axis_name="c", subcore_axis_name="s")
    tpn = N // _NW

    CH = min(32, tpn)
    NCH = tpn // CH
    _disp_scratch = ([pltpu.VMEM((CH, D), jnp.float32) for _ in range(2)]
                     + [pltpu.VMEM((CH,), jnp.int32) for _ in range(2 * NCH)]
                     + [pltpu.VMEM((CH,), jnp.float32) for _ in range(2 * NCH)]
                     + [pltpu.SemaphoreType.DMA])

    @functools.partial(
        pl.kernel,
        out_type=(jax.ShapeDtypeStruct((PEXP, D), jnp.float32),
                  jax.ShapeDtypeStruct((PEXP,), jnp.float32)),
        mesh=mesh,
        scratch_types=_disp_scratch,
    )
    def _sc_dispatch(post_h, valt_h, x_h, xs_h, vs_h, r0, r1, *bufs):
        pb = bufs[:2 * NCH]
        vb = bufs[2 * NCH:4 * NCH]
        sem = bufs[4 * NCH]
        rows = (r0, r1)
        wid = lax.axis_index("s") * _NC + lax.axis_index("c")
        base = wid * tpn
        for k in range(2):
            for ci in range(NCH):
                j = k * NCH + ci
                sl = pl.ds(base + ci * CH, CH)
                pltpu.sync_copy(post_h.at[k, sl], pb[j])
                pltpu.sync_copy(valt_h.at[k, sl], vb[j])
                pltpu.sync_copy(vb[j], vs_h.at[pb[j]])
        pend = []
        for ci in range(NCH):
            b = rows[ci % 2]
            if len(pend) >= 4:
                pend.pop(0).wait()
                pend.pop(0).wait()
            pltpu.sync_copy(x_h.at[pl.ds(base + ci * CH, CH)], b)
            pend.append(pltpu.async_copy(b, xs_h.at[pb[ci]], sem))
            pend.append(pltpu.async_copy(b, xs_h.at[pb[NCH + ci]], sem))
        for p in pend:
            p.wait()

    xs, vsort = _sc_dispatch(post, valt, xf)


    be_arr = misc[0, :NBMAX]
    nbt_arr = misc[0, 64:65]
    yexp = pl.pallas_call(
        _expert_body,
        grid_spec=pltpu.PrefetchScalarGridSpec(
            num_scalar_prefetch=2,
            grid=(NBMAX, 2),
            in_specs=[
                pl.BlockSpec((BT, D), lambda b, f, be, nbt: (b, 0)),
                pl.BlockSpec((1, FB, D), lambda b, f, be, nbt: (be[b], f, 0)),
                pl.BlockSpec((1, 1, FB), lambda b, f, be, nbt: (be[b], 0, f)),
                pl.BlockSpec((1, D, FB), lambda b, f, be, nbt: (be[b], 0, f)),
                pl.BlockSpec((1, 1, D), lambda b, f, be, nbt: (be[b], 0, 0)),
                pl.BlockSpec((1, 1, BT), lambda b, f, be, nbt: (b, 0, 0)),
            ],
            out_specs=pl.BlockSpec((BT, D), lambda b, f, be, nbt: (b, 0)),
        ),
        out_shape=jax.ShapeDtypeStruct((PEXP, D), jnp.float32),
        compiler_params=pltpu.CompilerParams(
            dimension_semantics=("arbitrary", "arbitrary")),
    )(be_arr, nbt_arr, xs, W1, b1.reshape(E, 1, F), W2,
      b2.reshape(E, 1, D), vsort.reshape(NBMAX, 1, BT))

    C3 = min(32, tpn)

    @functools.partial(
        pl.kernel,
        out_type=jax.ShapeDtypeStruct((N, D), jnp.float32),
        mesh=mesh,
        scratch_types=[pltpu.VMEM((C3, D), jnp.float32),
                       pltpu.VMEM((C3, D), jnp.float32),
                       pltpu.VMEM((C3, D), jnp.float32),
                       pltpu.VMEM((C3,), jnp.int32),
                       pltpu.VMEM((C3,), jnp.int32),
                       pltpu.SemaphoreType.DMA],
    )
    def _sc_combine(ysh_h, yexp_h, post_h, out_h, bsh, b0, b1, i0, i1, sem):
        wid = lax.axis_index("s") * _NC + lax.axis_index("c")
        for ci in range(tpn // C3):
            base = wid * tpn + ci * C3
            pltpu.sync_copy(ysh_h.at[pl.ds(base, C3)], bsh)
            pltpu.sync_copy(post_h.at[0, pl.ds(base, C3)], i0)
            pltpu.sync_copy(post_h.at[1, pl.ds(base, C3)], i1)
            pltpu.async_copy(yexp_h.at[i0], b0, sem).wait()
            pltpu.async_copy(yexp_h.at[i1], b1, sem).wait()

            def _vadd_row(r, carry):
                for c in range(D // 16):
                    s = pl.ds(c * 16, 16)
                    bsh[r, s] = bsh[r, s] + b0[r, s] + b1[r, s]
                return carry

            lax.fori_loop(0, C3, _vadd_row, 0)
            pltpu.sync_copy(bsh, out_h.at[pl.ds(base, C3)])

    out = _sc_combine(ysh, yexp, post)
    return out.reshape(B, T, D)


# speedup vs baseline: 2.4528x; 1.0534x over previous
import functools
import math

import jax
import jax.numpy as jnp
from jax import lax
from jax.experimental import pallas as pl
from jax.experimental.pallas import tpu as pltpu
from jax.experimental.pallas import tpu_sc as plsc

_NC, _NS, _NW = 2, 16, 32


def _excl_cumsum_rows(a):
    n = a.shape[0]
    s = jnp.concatenate([jnp.zeros_like(a[:1]), a[:-1]], axis=0)
    d = 1
    while d < n:
        s = s + jnp.concatenate([jnp.zeros_like(s[:d]), s[:-d]], axis=0)
        d *= 2
    return s


def _excl_cumsum_lanes(a):
    n = a.shape[1]
    s = jnp.concatenate([jnp.zeros_like(a[:, :1]), a[:, :-1]], axis=1)
    d = 1
    while d < n:
        s = s + jnp.concatenate([jnp.zeros_like(s[:, :d]), s[:, :-d]], axis=1)
        d *= 2
    return s


def _sel_lane(m, a, i16):
    return jnp.sum(jnp.where(i16 == a, m, 0.0), axis=1, keepdims=True)


def _route_body(x_ref, rw_ref, post_ref, valt_ref, misc_ref, *,
                n_exp, scale, bt, nbmax):
    x = x_ref[...]
    rw = rw_ref[...]
    logits = lax.dot_general(x, rw, (((1,), (1,)), ((), ())))
    i16 = lax.broadcasted_iota(jnp.int32, logits.shape, 1)
    logits = jnp.where(i16 < n_exp, logits, -1e30)
    p = jax.nn.softmax(logits, axis=-1)
    v1 = jnp.max(p, axis=-1, keepdims=True)
    a1 = jnp.argmax(p, axis=-1)[:, None]
    m1 = i16 == a1
    p2 = jnp.where(m1, -1.0, p)
    v2 = jnp.max(p2, axis=-1, keepdims=True)
    a2 = jnp.argmax(p2, axis=-1)[:, None]
    m2 = i16 == a2

    oh1 = m1.astype(jnp.float32)
    oh2 = m2.astype(jnp.float32)
    c1 = _excl_cumsum_rows(oh1)
    c2 = _excl_cumsum_rows(oh2)
    cnt1 = jnp.sum(oh1, axis=0, keepdims=True)
    cnt2 = jnp.sum(oh2, axis=0, keepdims=True)
    g = cnt1 + cnt2
    nb = jnp.ceil(g / float(bt))
    nboff = _excl_cumsum_lanes(nb)
    off = nboff * float(bt)

    pos0 = _sel_lane(off + c1, a1, i16)
    pos1 = _sel_lane(off + cnt1 + c2, a2, i16)
    post = jnp.concatenate([pos0, pos1], axis=1).astype(jnp.int32)
    post_ref[...] = jnp.transpose(post, (1, 0))
    valt = scale * jnp.concatenate([v1, v2], axis=1)
    valt_ref[...] = jnp.transpose(valt, (1, 0))

    r16 = lax.broadcasted_iota(jnp.int32, (16, 16), 0)
    l16 = lax.broadcasted_iota(jnp.int32, (16, 16), 1)
    nboff_col = jnp.sum(jnp.where(r16 == l16,
                                  jnp.broadcast_to(nboff, (16, 16)), 0.0),
                        axis=1, keepdims=True)
    bi = lax.broadcasted_iota(jnp.int32, (16, 128), 1)
    r128 = lax.broadcasted_iota(jnp.int32, (16, 128), 0)
    cmp = (bi >= nboff_col.astype(jnp.int32)) & (r128 < n_exp)
    be = jnp.sum(cmp.astype(jnp.int32), axis=0, keepdims=True) - 1
    be = jnp.clip(be, 0, n_exp - 1)
    nbt = jnp.sum(nb).astype(jnp.int32)
    l128 = lax.broadcasted_iota(jnp.int32, (1, 128), 1)
    misc = jnp.where(l128 < nbmax, be, 0)
    misc_ref[...] = jnp.where(l128 == 64, nbt, misc)


def _shared_body(x_ref, w1_ref, b1_ref, w2_ref, b2_ref, o_ref, *, scale):
    f = pl.program_id(1)
    x = x_ref[...]
    h = lax.dot_general(x, w1_ref[...], (((1,), (1,)), ((), ())),
                        preferred_element_type=jnp.float32)
    h = jax.nn.gelu(h + b1_ref[...]) * scale
    y = lax.dot_general(h, w2_ref[...], (((1,), (1,)), ((), ())),
                        preferred_element_type=jnp.float32)

    @pl.when(f == 0)
    def _init():
        o_ref[...] = jnp.zeros_like(o_ref) + scale * b2_ref[...]

    o_ref[...] += y


def _expert_body(be_ref, nbt_ref, xs_ref, w1_ref, b1_ref, w2_ref, b2_ref,
                 vs_ref, y_ref):
    b = pl.program_id(0)
    f = pl.program_id(1)

    @pl.when(b < nbt_ref[0])
    def _go():
        x = xs_ref[...]
        h = lax.dot_general(x, w1_ref[0], (((1,), (1,)), ((), ())),
                            preferred_element_type=jnp.float32)
        h = jax.nn.gelu(h + b1_ref[0])
        vcol = jnp.transpose(vs_ref[0], (1, 0))
        hs = h * vcol
        y = lax.dot_general(hs, w2_ref[0], (((1,), (1,)), ((), ())),
                            preferred_element_type=jnp.float32)

        @pl.when(f == 0)
        def _init():
            y_ref[...] = jnp.zeros_like(y_ref) + vcol * b2_ref[0]

        y_ref[...] += y


def kernel(x, router_w, W1, b1, W2, b2, Ws1, bs1, Ws2, bs2):
    B, T, D = x.shape
    E, F, _ = W1.shape
    N = B * T
    K = 2
    scale = 1.0 / math.sqrt(1.0 + K / E)
    BT = 512 if N >= 4096 else 128
    NBMAX = (N * K) // BT + E
    PEXP = NBMAX * BT
    FB = F // 2

    xf = x.reshape(N, D)
    rw16 = jnp.zeros((16, D), jnp.float32).at[:E].set(router_w)

    post, valt, misc = pl.pallas_call(
        functools.partial(_route_body, n_exp=E, scale=scale, bt=BT,
                          nbmax=NBMAX),
        grid=(1,),
        in_specs=[
            pl.BlockSpec((N, D), lambda i: (0, 0)),
            pl.BlockSpec((16, D), lambda i: (0, 0)),
        ],
        out_specs=[
            pl.BlockSpec((2, N), lambda i: (0, 0)),
            pl.BlockSpec((2, N), lambda i: (0, 0)),
            pl.BlockSpec((1, 128), lambda i: (0, 0)),
        ],
        out_shape=[
            jax.ShapeDtypeStruct((2, N), jnp.int32),
            jax.ShapeDtypeStruct((2, N), jnp.float32),
            jax.ShapeDtypeStruct((1, 128), jnp.int32),
        ],
    )(xf, rw16)

    ysh = pl.pallas_call(
        functools.partial(_shared_body, scale=scale),
        grid=(N // BT, 2),
        in_specs=[
            pl.BlockSpec((BT, D), lambda t, f: (t, 0)),
            pl.BlockSpec((FB, D), lambda t, f: (f, 0)),
            pl.BlockSpec((1, FB), lambda t, f: (0, f)),
            pl.BlockSpec((D, FB), lambda t, f: (0, f)),
            pl.BlockSpec((1, D), lambda t, f: (0, 0)),
        ],
        out_specs=pl.BlockSpec((BT, D), lambda t, f: (t, 0)),
        out_shape=jax.ShapeDtypeStruct((N, D), jnp.float32),
        compiler_params=pltpu.CompilerParams(
            dimension_semantics=("arbitrary", "arbitrary")),
    )(xf, Ws1, bs1.reshape(1, F), Ws2, bs2.reshape(1, D))

    mesh = plsc.VectorSubcoreMesh(core_axis_name="c", subcore_axis_name="s")
    tpn = N // _NW

    CH = min(32, tpn)
    NCH = tpn // CH
    _disp_scratch = ([pltpu.VMEM((CH, D), jnp.float32) for _ in range(2)]
                     + [pltpu.VMEM((CH,), jnp.int32) for _ in range(2 * NCH)]
                     + [pltpu.VMEM((CH,), jnp.float32) for _ in range(2 * NCH)]
                     + [pltpu.SemaphoreType.DMA])

    @functools.partial(
        pl.kernel,
        out_type=(jax.ShapeDtypeStruct((PEXP, D), jnp.float32),
                  jax.ShapeDtypeStruct((PEXP,), jnp.float32)),
        mesh=mesh,
        scratch_types=_disp_scratch,
    )
    def _sc_dispatch(post_h, valt_h, x_h, xs_h, vs_h, r0, r1, *bufs):
        pb = bufs[:2 * NCH]
        vb = bufs[2 * NCH:4 * NCH]
        sem = bufs[4 * NCH]
        rows = (r0, r1)
        wid = lax.axis_index("s") * _NC + lax.axis_index("c")
        base = wid * tpn
        small = []
        for k in range(2):
            for ci in range(NCH):
                j = k * NCH + ci
                sl = pl.ds(base + ci * CH, CH)
                small.append(pltpu.async_copy(post_h.at[k, sl], pb[j], sem))
                small.append(pltpu.async_copy(valt_h.at[k, sl], vb[j], sem))
        for p in small:
            p.wait()
        vsc = [pltpu.async_copy(vb[j], vs_h.at[pb[j]], sem)
               for j in range(2 * NCH)]
        pend = []
        for ci in range(NCH):
            b = rows[ci % 2]
            if len(pend) >= 4:
                pend.pop(0).wait()
                pend.pop(0).wait()
            pltpu.sync_copy(x_h.at[pl.ds(base + ci * CH, CH)], b)
            pend.append(pltpu.async_copy(b, xs_h.at[pb[ci]], sem))
            pend.append(pltpu.async_copy(b, xs_h.at[pb[NCH + ci]], sem))
        for p in pend + vsc:
            p.wait()

    xs, vsort = _sc_dispatch(post, valt, xf)


    be_arr = misc[0, :NBMAX]
    nbt_arr = misc[0, 64:65]
    yexp = pl.pallas_call(
        _expert_body,
        grid_spec=pltpu.PrefetchScalarGridSpec(
            num_scalar_prefetch=2,
            grid=(NBMAX, 2),
            in_specs=[
                pl.BlockSpec((BT, D), lambda b, f, be, nbt: (b, 0)),
                pl.BlockSpec((1, FB, D), lambda b, f, be, nbt: (be[b], f, 0)),
                pl.BlockSpec((1, 1, FB), lambda b, f, be, nbt: (be[b], 0, f)),
                pl.BlockSpec((1, D, FB), lambda b, f, be, nbt: (be[b], 0, f)),
                pl.BlockSpec((1, 1, D), lambda b, f, be, nbt: (be[b], 0, 0)),
                pl.BlockSpec((1, 1, BT), lambda b, f, be, nbt: (b, 0, 0)),
            ],
            out_specs=pl.BlockSpec((BT, D), lambda b, f, be, nbt: (b, 0)),
        ),
        out_shape=jax.ShapeDtypeStruct((PEXP, D), jnp.float32),
        compiler_params=pltpu.CompilerParams(
            dimension_semantics=("arbitrary", "arbitrary")),
    )(be_arr, nbt_arr, xs, W1, b1.reshape(E, 1, F), W2,
      b2.reshape(E, 1, D), vsort.reshape(NBMAX, 1, BT))

    C3 = min(16, tpn)
    NC3 = tpn // C3
    _cmb_scratch = ([pltpu.VMEM((C3, D), jnp.float32) for _ in range(6)]
                    + [pltpu.VMEM((C3,), jnp.int32) for _ in range(4)]
                    + [pltpu.SemaphoreType.DMA])

    @functools.partial(
        pl.kernel,
        out_type=jax.ShapeDtypeStruct((N, D), jnp.float32),
        mesh=mesh,
        scratch_types=_cmb_scratch,
    )
    def _sc_combine(ysh_h, yexp_h, post_h, out_h, *bufs):
        bset = (bufs[0:3], bufs[3:6])
        iset = (bufs[6:8], bufs[8:10])
        sem = bufs[10]
        wid = lax.axis_index("s") * _NC + lax.axis_index("c")

        def _prefetch(ci):
            base = wid * tpn + ci * C3
            bsh, b0, b1 = bset[ci % 2]
            i0, i1 = iset[ci % 2]
            pltpu.sync_copy(post_h.at[0, pl.ds(base, C3)], i0)
            pltpu.sync_copy(post_h.at[1, pl.ds(base, C3)], i1)
            return [pltpu.async_copy(ysh_h.at[pl.ds(base, C3)], bsh, sem),
                    pltpu.async_copy(yexp_h.at[i0], b0, sem),
                    pltpu.async_copy(yexp_h.at[i1], b1, sem)]

        pend_in = _prefetch(0)
        pend_out = []
        for ci in range(NC3):
            if pend_out:
                pend_out.pop(0).wait()
            nxt = _prefetch(ci + 1) if ci + 1 < NC3 else []
            for p in pend_in:
                p.wait()
            pend_in = nxt
            bsh, b0, b1 = bset[ci % 2]

            def _vadd_row(r, carry):
                for c in range(D // 16):
                    s = pl.ds(c * 16, 16)
                    bsh[r, s] = bsh[r, s] + b0[r, s] + b1[r, s]
                return carry

            lax.fori_loop(0, C3, _vadd_row, 0)
            base = wid * tpn + ci * C3
            pend_out.append(
                pltpu.async_copy(bsh, out_h.at[pl.ds(base, C3)], sem))
        for p in pend_out:
            p.wait()

    out = _sc_combine(ysh, yexp, post)
    return out.reshape(B, T, D)
